# baseline (speedup 1.0000x reference)
import jax
import jax.numpy as jnp
from jax import lax
from jax.experimental import pallas as pl
from jax.experimental.pallas import tpu as pltpu
from jax.experimental.pallas import tpu_sc as plsc

ROWS = 128
COLS = 32768
LANES = 16
NUM_CORES = 2
NUM_SUBCORES = 16
NUM_WORKERS = NUM_CORES * NUM_SUBCORES
ROWS_PER_WORKER = ROWS // NUM_WORKERS
NCHUNKS = COLS // LANES
NBINS = 512
BINW = 1.0 / NBINS
HCHUNKS = NBINS // LANES
NEWTON_STEPS = 4
U = 8


def _lane_sum(v):
    return plsc.cumsum(v)[LANES - 1]


def _lane_max(v):
    return plsc.cummax(v)[LANES - 1]


def _process_row(xv, cval, hcnt, hsum):
    ones = jnp.ones((LANES,), jnp.float32)
    zeros = jnp.zeros((LANES,), jnp.float32)

    UM = 16

    @plsc.parallel_loop(0, NCHUNKS, step=UM, carry=(zeros - jnp.inf,) * UM)
    def max_loop(i, ms):
        return tuple(
            jnp.maximum(ms[u], xv[pl.ds((i + u) * LANES, LANES)])
            for u in range(UM)
        )

    m = max_loop[0]
    for u in range(1, UM):
        m = jnp.maximum(m, max_loop[u])
    lo = _lane_max(m) - 1.0

    @plsc.parallel_loop(0, HCHUNKS, step=U)
    def _(i):
        for u in range(U):
            hcnt[pl.ds((i + u) * LANES, LANES)] = zeros
            hsum[pl.ds((i + u) * LANES, LANES)] = zeros

    lane_i = lax.iota(jnp.int32, LANES)

    @plsc.parallel_loop(0, NCHUNKS, step=1, unroll=U, carry=lane_i)
    def pos_v(i, pos):
        d = xv[pl.ds(i * LANES, LANES)] - lo
        pcnt = plsc.all_reduce_population_count(d > 0.0)
        keep = pcnt > 0
        plsc.store_scatter(cval, [pos], d, mask=keep)
        return pos + jnp.where(keep, LANES, 0)

    ncand = pos_v[0]
    cval[pl.ds(ncand, LANES)] = zeros
    ncv = lax.shift_right_logical(ncand + (LANES - 1), 4)

    def hist_body(i, _):
        d = cval[pl.ds(i * LANES, LANES)]
        binf = jnp.clip(d * float(NBINS), 0.0, float(NBINS - 1))
        idx = binf.astype(jnp.int32)
        msk = d > 0.0
        plsc.addupdate_scatter(hcnt, [idx], ones, mask=msk)
        plsc.addupdate_scatter(hsum, [idx], d, mask=msk)
        return 0

    lax.fori_loop(0, ncv, hist_body, 0)

    lane_f = lax.iota(jnp.int32, LANES).astype(jnp.float32)

    def scan_body(i, carry):
        ck, cs, best = carry
        cc = HCHUNKS - 1 - i
        c = hcnt[pl.ds(cc * LANES, LANES)]
        s = hsum[pl.ds(cc * LANES, LANES)]
        suf_c = lax.rev(plsc.cumsum(lax.rev(c, (0,))), (0,))
        suf_s = lax.rev(plsc.cumsum(lax.rev(s, (0,))), (0,))
        tot_c = suf_c[0]
        tot_s = suf_s[0]
        jf = lane_f + (cc * LANES).astype(jnp.float32)
        theta = (jf - 2.0) * BINW
        g = (suf_s + cs) - (suf_c + ck) * theta - 1.0
        cand = jnp.where(g > 0.0, jf, -1.0)
        best = jnp.maximum(best, cand)
        return ck + tot_c, cs + tot_s, best

    _, _, best_v = lax.fori_loop(
        0, HCHUNKS, scan_body,
        (jnp.float32(0.0), jnp.float32(0.0), jnp.full((LANES,), -1.0, jnp.float32)),
    )
    best = _lane_max(best_v)
    taup = jnp.full((LANES,), 1.0, jnp.float32) * jnp.maximum((best - 2.0) * BINW, 0.0)

    def newton(taup):
        def stat_body(i, carry):
            s, c = carry
            r = jnp.maximum(cval[pl.ds(i * LANES, LANES)] - taup, 0.0)
            return s + r, c + jnp.where(r > 0.0, 1.0, 0.0)

        s, c = lax.fori_loop(0, ncv, stat_body, (zeros, zeros))
        S = _lane_sum(s)
        K = _lane_sum(c)
        Sv = jnp.full((LANES,), 1.0, jnp.float32) * S
        Kv = jnp.full((LANES,), 1.0, jnp.float32) * K
        return taup + (Sv - 1.0) / jnp.maximum(Kv, 1.0)

    for _ in range(NEWTON_STEPS):
        taup = newton(taup)

    return lo + taup


def _row_output(xv, thr):
    @plsc.parallel_loop(0, NCHUNKS, step=16)
    def _(i):
        for u in range(16):
            sl = pl.ds((i + u) * LANES, LANES)
            xv[sl] = jnp.maximum(xv[sl] - thr, 0.0)


def _sc_body(x_hbm, out_hbm, xa, xb, cval, hcnt, hsum, sin0, sin1, sout0, sout1):
    cid = lax.axis_index("c")
    sid = lax.axis_index("s")
    wid = sid * NUM_CORES + cid
    base = wid * ROWS_PER_WORKER
    bufs = (xa, xb)
    sins = (sin0, sin1)
    souts = (sout0, sout1)
    pend_in = [None, None]
    pend_out = [None, None]
    pend_in[0] = pltpu.async_copy(x_hbm.at[base], xa, sins[0])
    for r in range(ROWS_PER_WORKER):
        b = r % 2
        pend_in[b].wait()
        thr = _process_row(bufs[b], cval, hcnt, hsum)
        if r + 1 < ROWS_PER_WORKER:
            nb = (r + 1) % 2
            if pend_out[nb] is not None:
                pend_out[nb].wait()
                pend_out[nb] = None
            pend_in[nb] = pltpu.async_copy(
                x_hbm.at[base + r + 1], bufs[nb], sins[nb]
            )
        _row_output(bufs[b], thr)
        pend_out[b] = pltpu.async_copy(bufs[b], out_hbm.at[base + r], souts[b])
    for b in (0, 1):
        if pend_out[b] is not None:
            pend_out[b].wait()


@jax.jit
def kernel(input):
    mesh = plsc.VectorSubcoreMesh(
        core_axis_name="c",
        subcore_axis_name="s",
        num_cores=NUM_CORES,
        num_subcores=NUM_SUBCORES,
    )
    run = pl.kernel(
        _sc_body,
        out_type=jax.ShapeDtypeStruct((ROWS, COLS), jnp.float32),
        mesh=mesh,
        compiler_params=pltpu.CompilerParams(needs_layout_passes=False),
        scratch_types=[
            pltpu.VMEM((COLS,), jnp.float32),
            pltpu.VMEM((COLS,), jnp.float32),
            pltpu.VMEM((COLS + LANES,), jnp.float32),
            pltpu.VMEM((NBINS,), jnp.float32),
            pltpu.VMEM((NBINS,), jnp.float32),
            pltpu.SemaphoreType.DMA,
            pltpu.SemaphoreType.DMA,
            pltpu.SemaphoreType.DMA,
            pltpu.SemaphoreType.DMA,
        ],
    )
    return run(input)


# speedup vs baseline: 1.0060x; 1.0060x over previous
import jax
import jax.numpy as jnp
from jax import lax
from jax.experimental import pallas as pl
from jax.experimental.pallas import tpu as pltpu
from jax.experimental.pallas import tpu_sc as plsc

ROWS = 128
COLS = 32768
LANES = 16
NUM_CORES = 2
NUM_SUBCORES = 16
NUM_WORKERS = NUM_CORES * NUM_SUBCORES
ROWS_PER_WORKER = ROWS // NUM_WORKERS
NCHUNKS = COLS // LANES
NBINS = 512
BINW = 1.0 / NBINS
HCHUNKS = NBINS // LANES
NEWTON_STEPS = 4
U = 8


def _lane_sum(v):
    return plsc.cumsum(v)[LANES - 1]


def _lane_max(v):
    return plsc.cummax(v)[LANES - 1]


def _process_row(xv, cval, hcnt, hsum):
    ones = jnp.ones((LANES,), jnp.float32)
    zeros = jnp.zeros((LANES,), jnp.float32)

    @plsc.parallel_loop(0, NCHUNKS, step=U, carry=(zeros - jnp.inf,) * U)
    def max_loop(i, ms):
        return tuple(
            jnp.maximum(ms[u], xv[pl.ds((i + u) * LANES, LANES)])
            for u in range(U)
        )

    m = max_loop[0]
    for u in range(1, U):
        m = jnp.maximum(m, max_loop[u])
    lo = _lane_max(m) - 1.0

    @plsc.parallel_loop(0, HCHUNKS, step=U)
    def _(i):
        for u in range(U):
            hcnt[pl.ds((i + u) * LANES, LANES)] = zeros
            hsum[pl.ds((i + u) * LANES, LANES)] = zeros

    lane_i = lax.iota(jnp.int32, LANES)

    @plsc.parallel_loop(0, NCHUNKS, step=1, unroll=U, carry=lane_i)
    def pos_v(i, pos):
        d = xv[pl.ds(i * LANES, LANES)] - lo
        pcnt = plsc.all_reduce_population_count(d > 0.0)
        keep = pcnt > 0
        plsc.store_scatter(cval, [pos], d, mask=keep)
        return pos + jnp.where(keep, LANES, 0)

    ncand = pos_v[0]
    cval[pl.ds(ncand, LANES)] = zeros
    ncv = lax.shift_right_logical(ncand + (LANES - 1), 4)

    def hist_body(i, _):
        d = cval[pl.ds(i * LANES, LANES)]
        binf = jnp.clip(d * float(NBINS), 0.0, float(NBINS - 1))
        idx = binf.astype(jnp.int32)
        msk = d > 0.0
        plsc.addupdate_scatter(hcnt, [idx], ones, mask=msk)
        plsc.addupdate_scatter(hsum, [idx], d, mask=msk)
        return 0

    lax.fori_loop(0, ncv, hist_body, 0)

    lane_f = lax.iota(jnp.int32, LANES).astype(jnp.float32)

    def scan_body(i, carry):
        ck, cs, best = carry
        cc = HCHUNKS - 1 - i
        c = hcnt[pl.ds(cc * LANES, LANES)]
        s = hsum[pl.ds(cc * LANES, LANES)]
        suf_c = lax.rev(plsc.cumsum(lax.rev(c, (0,))), (0,))
        suf_s = lax.rev(plsc.cumsum(lax.rev(s, (0,))), (0,))
        tot_c = suf_c[0]
        tot_s = suf_s[0]
        jf = lane_f + (cc * LANES).astype(jnp.float32)
        theta = (jf - 2.0) * BINW
        g = (suf_s + cs) - (suf_c + ck) * theta - 1.0
        cand = jnp.where(g > 0.0, jf, -1.0)
        best = jnp.maximum(best, cand)
        return ck + tot_c, cs + tot_s, best

    _, _, best_v = lax.fori_loop(
        0, HCHUNKS, scan_body,
        (jnp.float32(0.0), jnp.float32(0.0), jnp.full((LANES,), -1.0, jnp.float32)),
    )
    best = _lane_max(best_v)
    taup = jnp.full((LANES,), 1.0, jnp.float32) * jnp.maximum((best - 2.0) * BINW, 0.0)

    def newton(taup):
        def stat_body(i, carry):
            s, c = carry
            r = jnp.maximum(cval[pl.ds(i * LANES, LANES)] - taup, 0.0)
            return s + r, c + jnp.where(r > 0.0, 1.0, 0.0)

        s, c = lax.fori_loop(0, ncv, stat_body, (zeros, zeros))
        S = _lane_sum(s)
        K = _lane_sum(c)
        Sv = jnp.full((LANES,), 1.0, jnp.float32) * S
        Kv = jnp.full((LANES,), 1.0, jnp.float32) * K
        return taup + (Sv - 1.0) / jnp.maximum(Kv, 1.0)

    for _ in range(NEWTON_STEPS):
        taup = newton(taup)

    return lo + taup


def _row_output(xv, thr):
    @plsc.parallel_loop(0, NCHUNKS, step=U)
    def _(i):
        for u in range(U):
            sl = pl.ds((i + u) * LANES, LANES)
            xv[sl] = jnp.maximum(xv[sl] - thr, 0.0)


def _sc_body(x_hbm, out_hbm, xa, xb, cval, hcnt, hsum, sin0, sin1, sout0, sout1):
    cid = lax.axis_index("c")
    sid = lax.axis_index("s")
    wid = sid * NUM_CORES + cid
    base = wid * ROWS_PER_WORKER
    bufs = (xa, xb)
    sins = (sin0, sin1)
    souts = (sout0, sout1)
    pend_in = [None, None]
    pend_out = [None, None]
    pend_in[0] = pltpu.async_copy(x_hbm.at[base], xa, sins[0])
    for r in range(ROWS_PER_WORKER):
        b = r % 2
        pend_in[b].wait()
        thr = _process_row(bufs[b], cval, hcnt, hsum)
        if r + 1 < ROWS_PER_WORKER:
            nb = (r + 1) % 2
            if pend_out[nb] is not None:
                pend_out[nb].wait()
                pend_out[nb] = None
            pend_in[nb] = pltpu.async_copy(
                x_hbm.at[base + r + 1], bufs[nb], sins[nb]
            )
        _row_output(bufs[b], thr)
        pend_out[b] = pltpu.async_copy(bufs[b], out_hbm.at[base + r], souts[b])
    for b in (0, 1):
        if pend_out[b] is not None:
            pend_out[b].wait()


@jax.jit
def kernel(input):
    mesh = plsc.VectorSubcoreMesh(
        core_axis_name="c",
        subcore_axis_name="s",
        num_cores=NUM_CORES,
        num_subcores=NUM_SUBCORES,
    )
    run = pl.kernel(
        _sc_body,
        out_type=jax.ShapeDtypeStruct((ROWS, COLS), jnp.float32),
        mesh=mesh,
        compiler_params=pltpu.CompilerParams(needs_layout_passes=False),
        scratch_types=[
            pltpu.VMEM((COLS,), jnp.float32),
            pltpu.VMEM((COLS,), jnp.float32),
            pltpu.VMEM((COLS + LANES,), jnp.float32),
            pltpu.VMEM((NBINS,), jnp.float32),
            pltpu.VMEM((NBINS,), jnp.float32),
            pltpu.SemaphoreType.DMA,
            pltpu.SemaphoreType.DMA,
            pltpu.SemaphoreType.DMA,
            pltpu.SemaphoreType.DMA,
        ],
    )
    return run(input)
